# initial kernel scaffold (unmeasured)
import jax
import jax.numpy as jnp
from jax import lax
from jax.experimental import pallas as pl
from jax.experimental.pallas import tpu as pltpu

N_DEV = 16
N_EXP = 64
EXP_PER_DEV = N_EXP // N_DEV
CAP = 204
D_IN = 256
D_OUT = 512
TOK = 1024

_CompilerParams = getattr(pltpu, "CompilerParams", None) or getattr(
    pltpu, "TPUCompilerParams"
)


def kernel(x, router_W, route_idx, expert_W):
    del router_W
    xb = x.astype(jnp.bfloat16)
    wb = expert_W.astype(jnp.bfloat16)
    route = route_idx.astype(jnp.int32)

    def body(
        x_ref,
        route_ref,
        w_ref,
        out_ref,
        wexp_ref,
        cnts_ref,
        w_send,
        w_recv,
        c_send,
        c_recv,
    ):
        my = lax.axis_index("i")
        left = lax.rem(my + N_DEV - 1, N_DEV)
        right = lax.rem(my + 1, N_DEV)

        barrier = pltpu.get_barrier_semaphore()
        for nbr in (left, right):
            pl.semaphore_signal(
                barrier, inc=1, device_id=(nbr,), device_id_type=pl.DeviceIdType.MESH
            )
        pl.semaphore_wait(barrier, 2)

        route_tok = route_ref[...]
        eid = lax.broadcasted_iota(jnp.int32, (TOK, N_EXP), 1)
        onehot = (route_tok == eid).astype(jnp.float32)
        ii = lax.broadcasted_iota(jnp.int32, (TOK, TOK), 0)
        jj = lax.broadcasted_iota(jnp.int32, (TOK, TOK), 1)
        tril = (jj < ii).astype(jnp.float32)
        rank_f = jnp.dot(tril, onehot, preferred_element_type=jnp.float32)
        rank_tok = jnp.sum(rank_f * onehot, axis=1, keepdims=True).astype(jnp.int32)
        counts = jnp.sum(onehot, axis=0, keepdims=True).astype(jnp.int32)
        pad = lax.dynamic_update_slice(jnp.zeros((8, 128), jnp.int32), counts, (0, 0))

        wexp_ref[my] = w_ref[...]
        cnts_ref[my] = pad

        for h in range(N_DEV - 1):
            o_send = lax.rem(my - h + 2 * N_DEV, N_DEV)
            w_rdma = pltpu.make_async_remote_copy(
                src_ref=wexp_ref.at[o_send],
                dst_ref=wexp_ref.at[o_send],
                send_sem=w_send.at[h],
                recv_sem=w_recv.at[h],
                device_id=(right,),
                device_id_type=pl.DeviceIdType.MESH,
            )
            c_rdma = pltpu.make_async_remote_copy(
                src_ref=cnts_ref.at[o_send],
                dst_ref=cnts_ref.at[o_send],
                send_sem=c_send.at[h],
                recv_sem=c_recv.at[h],
                device_id=(right,),
                device_id_type=pl.DeviceIdType.MESH,
            )
            w_rdma.start()
            c_rdma.start()
            w_rdma.wait()
            c_rdma.wait()

        offs = jnp.zeros((8, 128), jnp.int32)
        for s in range(N_DEV):
            offs = offs + jnp.where(s < my, cnts_ref[s], 0)

        xv = x_ref[...]
        acc = jnp.zeros((TOK, D_OUT), jnp.float32)
        for e in range(N_EXP):
            keep = (route_tok == e) & (rank_tok + offs[0, e] < CAP)
            xm = jnp.where(keep, xv, 0)
            w_e = wexp_ref[e // EXP_PER_DEV, e % EXP_PER_DEV]
            acc = acc + jnp.dot(xm, w_e, preferred_element_type=jnp.float32)
        out_ref[...] = acc

    return pl.pallas_call(
        body,
        out_shape=jax.ShapeDtypeStruct((TOK, D_OUT), jnp.float32),
        in_specs=[pl.BlockSpec(memory_space=pltpu.VMEM)] * 3,
        out_specs=pl.BlockSpec(memory_space=pltpu.VMEM),
        scratch_shapes=[
            pltpu.VMEM((N_DEV, EXP_PER_DEV, D_IN, D_OUT), jnp.bfloat16),
            pltpu.VMEM((N_DEV, 8, 128), jnp.int32),
            pltpu.SemaphoreType.DMA((N_DEV - 1,)),
            pltpu.SemaphoreType.DMA((N_DEV - 1,)),
            pltpu.SemaphoreType.DMA((N_DEV - 1,)),
            pltpu.SemaphoreType.DMA((N_DEV - 1,)),
        ],
        compiler_params=_CompilerParams(collective_id=0),
    )(xb, route, wb)


# baseline (device time: 246092 ns/iter reference)
import jax
import jax.numpy as jnp
from jax import lax
from jax.experimental import pallas as pl
from jax.experimental.pallas import tpu as pltpu

N_DEV = 16
N_EXP = 64
EXP_PER_DEV = N_EXP // N_DEV
CAP = 204
D_IN = 256
D_OUT = 512
TOK = 1024

_CompilerParams = getattr(pltpu, "CompilerParams", None) or getattr(
    pltpu, "TPUCompilerParams"
)


def kernel(x, router_W, route_idx, expert_W):
    del router_W
    xb = x.astype(jnp.bfloat16)
    wb = expert_W.astype(jnp.bfloat16)
    route = route_idx.astype(jnp.int32)

    def body(
        x_ref,
        route_ref,
        w_ref,
        out_ref,
        wexp_ref,
        cnts_ref,
        w_send,
        w_recv,
        c_send,
        c_recv,
    ):
        my = lax.axis_index("i")
        left = lax.rem(my + N_DEV - 1, N_DEV)
        right = lax.rem(my + 1, N_DEV)

        barrier = pltpu.get_barrier_semaphore()
        for nbr in (left, right):
            pl.semaphore_signal(
                barrier, inc=1, device_id=(nbr,), device_id_type=pl.DeviceIdType.MESH
            )
        pl.semaphore_wait(barrier, 2)

        route_tok = route_ref[...]
        eid128 = lax.broadcasted_iota(jnp.int32, (TOK, 128), 1)
        oh128 = (route_tok == eid128).astype(jnp.bfloat16)
        onehot = oh128[:, 0:N_EXP]
        ii = lax.broadcasted_iota(jnp.int32, (TOK, TOK), 0)
        jj = lax.broadcasted_iota(jnp.int32, (TOK, TOK), 1)
        tril = (jj < ii).astype(jnp.bfloat16)
        rank_f = jnp.dot(tril, onehot, preferred_element_type=jnp.float32)
        counts = jnp.sum(oh128, axis=0, keepdims=True).astype(jnp.int32)
        row8 = lax.broadcasted_iota(jnp.int32, (8, 128), 0)
        pad = jnp.where(row8 == 0, jnp.broadcast_to(counts, (8, 128)), 0)

        wexp_ref[my] = w_ref[...]
        cnts_ref[my] = pad

        for h in range(N_DEV - 1):
            o_send = lax.rem(my - h + 2 * N_DEV, N_DEV)
            w_rdma = pltpu.make_async_remote_copy(
                src_ref=wexp_ref.at[o_send],
                dst_ref=wexp_ref.at[o_send],
                send_sem=w_send.at[h],
                recv_sem=w_recv.at[h],
                device_id=(right,),
                device_id_type=pl.DeviceIdType.MESH,
            )
            c_rdma = pltpu.make_async_remote_copy(
                src_ref=cnts_ref.at[o_send],
                dst_ref=cnts_ref.at[o_send],
                send_sem=c_send.at[h],
                recv_sem=c_recv.at[h],
                device_id=(right,),
                device_id_type=pl.DeviceIdType.MESH,
            )
            w_rdma.start()
            c_rdma.start()
            w_rdma.wait()
            c_rdma.wait()

        offs = jnp.zeros((8, 128), jnp.int32)
        for s in range(N_DEV):
            offs = offs + jnp.where(s < my, cnts_ref[s], 0)
        offs_row = offs[0:1, 0:N_EXP].astype(jnp.float32)

        under_cap = (rank_f + offs_row < float(CAP)).astype(jnp.bfloat16)
        keep_tok = jnp.sum(onehot * under_cap, axis=1, keepdims=True)
        keep_b = keep_tok > 0.5

        out_ref[...] = jnp.zeros((TOK, D_OUT), jnp.float32)

        def ebody(e, carry):
            s = lax.div(e, EXP_PER_DEV)
            k = lax.rem(e, EXP_PER_DEV)
            w_e = wexp_ref[s, k]
            m = jnp.logical_and(route_tok == e, keep_b)
            xm = jnp.where(m, x_ref[...], 0)
            out_ref[...] += jnp.dot(xm, w_e, preferred_element_type=jnp.float32)
            return carry

        lax.fori_loop(0, N_EXP, ebody, 0)

    return pl.pallas_call(
        body,
        out_shape=jax.ShapeDtypeStruct((TOK, D_OUT), jnp.float32),
        in_specs=[pl.BlockSpec(memory_space=pltpu.VMEM)] * 3,
        out_specs=pl.BlockSpec(memory_space=pltpu.VMEM),
        scratch_shapes=[
            pltpu.VMEM((N_DEV, EXP_PER_DEV, D_IN, D_OUT), jnp.bfloat16),
            pltpu.VMEM((N_DEV, 8, 128), jnp.int32),
            pltpu.SemaphoreType.DMA((N_DEV - 1,)),
            pltpu.SemaphoreType.DMA((N_DEV - 1,)),
            pltpu.SemaphoreType.DMA((N_DEV - 1,)),
            pltpu.SemaphoreType.DMA((N_DEV - 1,)),
        ],
        compiler_params=_CompilerParams(
            collective_id=0, vmem_limit_bytes=100 * 1024 * 1024
        ),
    )(xb, route, wb)


# device time: 168649 ns/iter; 1.4592x vs baseline; 1.4592x over previous
import jax
import jax.numpy as jnp
from jax import lax
from jax.experimental import pallas as pl
from jax.experimental.pallas import tpu as pltpu

N_DEV = 16
N_EXP = 64
EXP_PER_DEV = N_EXP // N_DEV
CAP = 204
D_IN = 256
D_OUT = 512
TOK = 1024

_CompilerParams = getattr(pltpu, "CompilerParams", None) or getattr(
    pltpu, "TPUCompilerParams"
)


def kernel(x, router_W, route_idx, expert_W):
    del router_W
    xb = x.astype(jnp.bfloat16)
    wb = expert_W.astype(jnp.bfloat16)
    route = route_idx.astype(jnp.int32)

    def body(
        x_ref,
        route_ref,
        w_ref,
        out_ref,
        wexp_ref,
        cnts_ref,
        w_send,
        w_recv,
        c_send,
        c_recv,
    ):
        my = lax.axis_index("i")
        left = lax.rem(my + N_DEV - 1, N_DEV)
        right = lax.rem(my + 1, N_DEV)

        barrier = pltpu.get_barrier_semaphore()
        for j in range(1, N_DEV):
            pl.semaphore_signal(
                barrier,
                inc=1,
                device_id=(lax.rem(my + j, N_DEV),),
                device_id_type=pl.DeviceIdType.MESH,
            )
        pl.semaphore_wait(barrier, N_DEV - 1)

        route_tok = route_ref[...]
        eid128 = lax.broadcasted_iota(jnp.int32, (TOK, 128), 1)
        oh128 = (route_tok == eid128).astype(jnp.bfloat16)
        onehot = oh128[:, 0:N_EXP]
        ii = lax.broadcasted_iota(jnp.int32, (TOK, TOK), 0)
        jj = lax.broadcasted_iota(jnp.int32, (TOK, TOK), 1)
        tril = (jj < ii).astype(jnp.bfloat16)
        rank_f = jnp.dot(tril, onehot, preferred_element_type=jnp.float32)
        counts = jnp.sum(oh128, axis=0, keepdims=True).astype(jnp.int32)
        row8 = lax.broadcasted_iota(jnp.int32, (8, 128), 0)
        pad = jnp.where(row8 == 0, jnp.broadcast_to(counts, (8, 128)), 0)

        wexp_ref[my] = w_ref[...]
        cnts_ref[my] = pad

        c_rd = []
        for j in range(N_DEV - 1):
            dest = lax.rem(my + 1 + j, N_DEV)
            rd = pltpu.make_async_remote_copy(
                src_ref=cnts_ref.at[my],
                dst_ref=cnts_ref.at[my],
                send_sem=c_send.at[j],
                recv_sem=c_recv.at[j],
                device_id=(dest,),
                device_id_type=pl.DeviceIdType.MESH,
            )
            rd.start()
            c_rd.append(rd)
        for rd in c_rd:
            rd.wait_recv()
        for rd in c_rd:
            rd.wait_send()

        def mk_ring(origin, slot, dest):
            return pltpu.make_async_remote_copy(
                src_ref=wexp_ref.at[origin],
                dst_ref=wexp_ref.at[origin],
                send_sem=w_send.at[slot],
                recv_sem=w_recv.at[slot],
                device_id=(dest,),
                device_id_type=pl.DeviceIdType.MESH,
            )

        offs = jnp.zeros((8, 128), jnp.int32)
        for s in range(N_DEV):
            offs = offs + jnp.where(s < my, cnts_ref[s], 0)
        offs_row = offs[0:1, 0:N_EXP].astype(jnp.float32)

        under_cap = (rank_f + offs_row < float(CAP)).astype(jnp.bfloat16)
        keep_tok = jnp.sum(onehot * under_cap, axis=1, keepdims=True)
        keep_b = keep_tok > 0.5

        out_ref[...] = jnp.zeros((TOK, D_OUT), jnp.float32)

        def gemm_chunk(o):
            def kbody(k, c):
                w_e = wexp_ref[o, k]
                m = jnp.logical_and(route_tok == o * EXP_PER_DEV + k, keep_b)
                xm = jnp.where(m, x_ref[...], 0)
                out_ref[...] += jnp.dot(xm, w_e, preferred_element_type=jnp.float32)
                return c

            lax.fori_loop(0, EXP_PER_DEV, kbody, 0)

        gemm_chunk(my)

        r_rds = [None] * 8
        l_rds = [None] * 7
        r_rds[0] = mk_ring(my, 0, right)
        l_rds[0] = mk_ring(my, 8, left)
        r_rds[0].start()
        l_rds[0].start()
        for h in range(8):
            r_rds[h].wait_recv()
            if h < 7:
                l_rds[h].wait_recv()
            o_r = lax.rem(my - h - 1 + 2 * N_DEV, N_DEV)
            o_l = lax.rem(my + h + 1, N_DEV)
            if h + 1 < 8:
                r_rds[h + 1] = mk_ring(o_r, h + 1, right)
                r_rds[h + 1].start()
            if h + 1 < 7:
                l_rds[h + 1] = mk_ring(o_l, 8 + h + 1, left)
                l_rds[h + 1].start()
            r_rds[h].wait_send()
            if h < 7:
                l_rds[h].wait_send()

        for h in range(8):
            gemm_chunk(lax.rem(my - h - 1 + 2 * N_DEV, N_DEV))
            if h < 7:
                gemm_chunk(lax.rem(my + h + 1, N_DEV))

    return pl.pallas_call(
        body,
        out_shape=jax.ShapeDtypeStruct((TOK, D_OUT), jnp.float32),
        in_specs=[pl.BlockSpec(memory_space=pltpu.VMEM)] * 3,
        out_specs=pl.BlockSpec(memory_space=pltpu.VMEM),
        scratch_shapes=[
            pltpu.VMEM((N_DEV, EXP_PER_DEV, D_IN, D_OUT), jnp.bfloat16),
            pltpu.VMEM((N_DEV, 8, 128), jnp.int32),
            pltpu.SemaphoreType.DMA((N_DEV - 1,)),
            pltpu.SemaphoreType.DMA((N_DEV - 1,)),
            pltpu.SemaphoreType.DMA((N_DEV - 1,)),
            pltpu.SemaphoreType.DMA((N_DEV - 1,)),
        ],
        compiler_params=_CompilerParams(
            collective_id=0, vmem_limit_bytes=100 * 1024 * 1024
        ),
    )(xb, route, wb)


# device time: 135351 ns/iter; 1.8182x vs baseline; 1.2460x over previous
import jax
import jax.numpy as jnp
from jax import lax
from jax.experimental import pallas as pl
from jax.experimental.pallas import tpu as pltpu

N_DEV = 16
N_EXP = 64
EXP_PER_DEV = N_EXP // N_DEV
CAP = 204
D_IN = 256
D_OUT = 512
TOK = 1024

_CompilerParams = getattr(pltpu, "CompilerParams", None) or getattr(
    pltpu, "TPUCompilerParams"
)


def kernel(x, router_W, route_idx, expert_W):
    del router_W
    xb = x.astype(jnp.bfloat16)
    wb = expert_W.astype(jnp.bfloat16)
    route = route_idx.astype(jnp.int32)

    def body(
        x_ref,
        route_ref,
        w_ref,
        out_ref,
        wexp_ref,
        cnts_ref,
        w_send,
        w_recv,
        c_send,
        c_recv,
    ):
        my = lax.axis_index("i")
        left = lax.rem(my + N_DEV - 1, N_DEV)
        right = lax.rem(my + 1, N_DEV)

        barrier = pltpu.get_barrier_semaphore()
        for j in range(1, N_DEV):
            pl.semaphore_signal(
                barrier,
                inc=1,
                device_id=(lax.rem(my + j, N_DEV),),
                device_id_type=pl.DeviceIdType.MESH,
            )
        pl.semaphore_wait(barrier, N_DEV - 1)

        route_tok = route_ref[...]
        eid128 = lax.broadcasted_iota(jnp.int32, (TOK, 128), 1)
        oh128 = (route_tok == eid128).astype(jnp.bfloat16)
        onehot = oh128[:, 0:N_EXP]
        counts = jnp.sum(oh128, axis=0, keepdims=True).astype(jnp.int32)
        row8 = lax.broadcasted_iota(jnp.int32, (8, 128), 0)
        pad = jnp.where(row8 == 0, jnp.broadcast_to(counts, (8, 128)), 0)

        wexp_ref[my] = w_ref[...]
        cnts_ref[my] = pad

        c_rd = []
        for j in range(N_DEV - 1):
            dest = lax.rem(my + 1 + j, N_DEV)
            rd = pltpu.make_async_remote_copy(
                src_ref=cnts_ref.at[my],
                dst_ref=cnts_ref.at[my],
                send_sem=c_send.at[j],
                recv_sem=c_recv.at[j],
                device_id=(dest,),
                device_id_type=pl.DeviceIdType.MESH,
            )
            rd.start()
            c_rd.append(rd)

        def mk_ring(origin, slot, dest):
            return pltpu.make_async_remote_copy(
                src_ref=wexp_ref.at[origin],
                dst_ref=wexp_ref.at[origin],
                send_sem=w_send.at[slot],
                recv_sem=w_recv.at[slot],
                device_id=(dest,),
                device_id_type=pl.DeviceIdType.MESH,
            )

        r_rds = [None] * 8
        l_rds = [None] * 7
        r_rds[0] = mk_ring(my, 0, right)
        l_rds[0] = mk_ring(my, 8, left)
        r_rds[0].start()
        l_rds[0].start()

        ii = lax.broadcasted_iota(jnp.int32, (TOK, TOK), 0)
        jj = lax.broadcasted_iota(jnp.int32, (TOK, TOK), 1)
        tril = (jj < ii).astype(jnp.bfloat16)
        rank_f = jnp.dot(tril, onehot, preferred_element_type=jnp.float32)

        for rd in c_rd:
            rd.wait_recv()
        for rd in c_rd:
            rd.wait_send()

        offs = jnp.zeros((8, 128), jnp.int32)
        for s in range(N_DEV):
            offs = offs + jnp.where(s < my, cnts_ref[s], 0)
        offs_row = offs[0:1, 0:N_EXP].astype(jnp.float32)

        under_cap = (rank_f + offs_row < float(CAP)).astype(jnp.bfloat16)
        keep_tok = jnp.sum(onehot * under_cap, axis=1, keepdims=True)
        keep_b = keep_tok > 0.5

        out_ref[...] = jnp.zeros((TOK, D_OUT), jnp.float32)

        def gemm_chunk(o):
            def kbody(k, c):
                w_e = wexp_ref[o, k]
                m = jnp.logical_and(route_tok == o * EXP_PER_DEV + k, keep_b)
                xm = jnp.where(m, x_ref[...], 0)
                out_ref[...] += jnp.dot(xm, w_e, preferred_element_type=jnp.float32)
                return c

            lax.fori_loop(0, EXP_PER_DEV, kbody, 0)

        gemm_chunk(my)

        for h in range(8):
            r_rds[h].wait_recv()
            if h < 7:
                l_rds[h].wait_recv()
            o_r = lax.rem(my - h - 1 + 2 * N_DEV, N_DEV)
            o_l = lax.rem(my + h + 1, N_DEV)
            if h + 1 < 8:
                r_rds[h + 1] = mk_ring(o_r, h + 1, right)
                r_rds[h + 1].start()
            if h + 1 < 7:
                l_rds[h + 1] = mk_ring(o_l, 8 + h + 1, left)
                l_rds[h + 1].start()
            r_rds[h].wait_send()
            if h < 7:
                l_rds[h].wait_send()
            gemm_chunk(o_r)
            if h < 7:
                gemm_chunk(o_l)

    return pl.pallas_call(
        body,
        out_shape=jax.ShapeDtypeStruct((TOK, D_OUT), jnp.float32),
        in_specs=[pl.BlockSpec(memory_space=pltpu.VMEM)] * 3,
        out_specs=pl.BlockSpec(memory_space=pltpu.VMEM),
        scratch_shapes=[
            pltpu.VMEM((N_DEV, EXP_PER_DEV, D_IN, D_OUT), jnp.bfloat16),
            pltpu.VMEM((N_DEV, 8, 128), jnp.int32),
            pltpu.SemaphoreType.DMA((N_DEV - 1,)),
            pltpu.SemaphoreType.DMA((N_DEV - 1,)),
            pltpu.SemaphoreType.DMA((N_DEV - 1,)),
            pltpu.SemaphoreType.DMA((N_DEV - 1,)),
        ],
        compiler_params=_CompilerParams(
            collective_id=0, vmem_limit_bytes=100 * 1024 * 1024
        ),
    )(xb, route, wb)
